# baseline (speedup 1.0000x reference)
import jax
import jax.numpy as jnp
from jax import lax
from jax.experimental import pallas as pl
from jax.experimental.pallas import tpu as pltpu
from jax.experimental.pallas import tpu_sc as plsc

VOCAB = 1000000
D = 64
B = 16384
NC = 2
NS = 16
NW = NC * NS
BPW = B // NW
LANES = 16


def _sc_body(table_hbm, idx1_hbm, idx2_hbm, vis1_hbm, vis2_hbm,
             out1_hbm, out2_hbm, idx_v, rows_v, vis_v, sem):
  wid = lax.axis_index("s") * NC + lax.axis_index("c")
  base = wid * BPW

  def one_lookup(idx_hbm, vis_hbm, out_hbm):
    pltpu.sync_copy(idx_hbm.at[pl.ds(base, BPW)], idx_v)
    gather = pltpu.async_copy(table_hbm.at[idx_v], rows_v, sem)
    pltpu.sync_copy(vis_hbm.at[pl.ds(base, BPW)], vis_v)
    gather.wait()

    def add_row(r, carry):
      for j in range(D // LANES):
        sl = pl.ds(j * LANES, LANES)
        rows_v[r, sl] = rows_v[r, sl] + vis_v[r, sl]
      return carry

    lax.fori_loop(0, BPW, add_row, 0)
    pltpu.sync_copy(rows_v, out_hbm.at[pl.ds(base, BPW)])

  one_lookup(idx1_hbm, vis1_hbm, out1_hbm)
  one_lookup(idx2_hbm, vis2_hbm, out2_hbm)


@jax.jit
def _run(vis1, vis2, idx1, idx2, table):
  mesh = plsc.VectorSubcoreMesh(
      core_axis_name="c", subcore_axis_name="s",
      num_cores=NC, num_subcores=NS)
  out1, out2 = pl.kernel(
      _sc_body,
      out_type=(jax.ShapeDtypeStruct((B, D), jnp.float32),
                jax.ShapeDtypeStruct((B, D), jnp.float32)),
      mesh=mesh,
      scratch_types=[
          pltpu.VMEM((BPW,), jnp.int32),
          pltpu.VMEM((BPW, D), jnp.float32),
          pltpu.VMEM((BPW, D), jnp.float32),
          pltpu.SemaphoreType.DMA,
      ],
      compiler_params=pltpu.CompilerParams(use_tc_tiling_on_sc=False),
  )(table, idx1, idx2, vis1, vis2)
  return out1.reshape(1, B * D), out2.reshape(1, B * D)


def kernel(vis_features_first, vis_features_second, inputs_first,
           inputs_second, embedding_table):
  vis1 = vis_features_first.reshape(B, D)
  vis2 = vis_features_second.reshape(B, D)
  idx1 = inputs_first.astype(jnp.int32)
  idx2 = inputs_second.astype(jnp.int32)
  return _run(vis1, vis2, idx1, idx2, embedding_table)


# speedup vs baseline: 1.0042x; 1.0042x over previous
import jax
import jax.numpy as jnp
from jax import lax
from jax.experimental import pallas as pl
from jax.experimental.pallas import tpu as pltpu
from jax.experimental.pallas import tpu_sc as plsc

VOCAB = 1000000
D = 64
B = 16384
NC = 2
NS = 16
NW = NC * NS
BPW = B // NW
LANES = 16


def _sc_body(table_hbm, idx1_hbm, idx2_hbm, vis1_hbm, vis2_hbm,
             out1_hbm, out2_hbm, idx_v, rows_v, acc_v, sem):
  wid = lax.axis_index("s") * NC + lax.axis_index("c")
  base = wid * BPW

  def one_lookup(idx_hbm, vis_hbm, out_hbm):
    pltpu.sync_copy(idx_hbm.at[pl.ds(base, BPW)], idx_v)
    gather = pltpu.async_copy(table_hbm.at[idx_v], rows_v, sem)
    pltpu.sync_copy(vis_hbm.at[0, pl.ds(base * D, BPW * D)], acc_v)
    gather.wait()

    def add_row(r, carry):
      for j in range(D // LANES):
        fsl = pl.ds(r * D + j * LANES, LANES)
        acc_v[fsl] = acc_v[fsl] + rows_v[r, pl.ds(j * LANES, LANES)]
      return carry

    lax.fori_loop(0, BPW, add_row, 0)
    pltpu.sync_copy(acc_v, out_hbm.at[0, pl.ds(base * D, BPW * D)])

  one_lookup(idx1_hbm, vis1_hbm, out1_hbm)
  one_lookup(idx2_hbm, vis2_hbm, out2_hbm)


@jax.jit
def _run(vis1, vis2, idx1, idx2, table):
  mesh = plsc.VectorSubcoreMesh(
      core_axis_name="c", subcore_axis_name="s",
      num_cores=NC, num_subcores=NS)
  return pl.kernel(
      _sc_body,
      out_type=(jax.ShapeDtypeStruct((1, B * D), jnp.float32),
                jax.ShapeDtypeStruct((1, B * D), jnp.float32)),
      mesh=mesh,
      scratch_types=[
          pltpu.VMEM((BPW,), jnp.int32),
          pltpu.VMEM((BPW, D), jnp.float32),
          pltpu.VMEM((BPW * D,), jnp.float32),
          pltpu.SemaphoreType.DMA,
      ],
      compiler_params=pltpu.CompilerParams(use_tc_tiling_on_sc=False),
  )(table, idx1, idx2, vis1, vis2)


def kernel(vis_features_first, vis_features_second, inputs_first,
           inputs_second, embedding_table):
  idx1 = inputs_first.astype(jnp.int32)
  idx2 = inputs_second.astype(jnp.int32)
  return _run(vis_features_first, vis_features_second, idx1, idx2,
              embedding_table)
